# baseline (speedup 1.0000x reference)
import jax
import jax.numpy as jnp
from jax import lax
from jax.experimental import pallas as pl
from jax.experimental.pallas import tpu as pltpu
from jax.experimental.pallas import tpu_sc as plsc

BS, AN, CN, OD = 4096, 200, 18, 32
NC, NS, L = 2, 16, 16
NW = NC * NS
BT = BS // NW
NA = 8
NCHUNK = AN // NA
NGRP = BT // L
EFS = CN * OD + 1
NXB = 3
NOB = 2


def _body(x_hbm, emb_hbm, out_hbm, x_v, ef_v, out_v, sem_in, sem_out):
    wid = lax.axis_index("c") * NS + lax.axis_index("s")
    b0 = wid * BT

    for i in range(CN):
        for h in range(OD // L):
            pltpu.sync_copy(emb_hbm.at[i, pl.ds(h * L, L)],
                            ef_v.at[pl.ds(i * OD + h * L, L)])
    for l in range(1, L):
        for w in range(CN * OD // L):
            ef_v[pl.ds(l * EFS + w * L, L)] = ef_v[pl.ds(w * L, L)]
    skew = lax.iota(jnp.int32, L) * EFS

    def in_src(ci):
        return x_hbm.at[:, pl.ds(ci * NA, NA), pl.ds(b0, BT)]

    def out_dst(ci):
        return out_hbm.at[pl.ds(ci * NA, NA), :, pl.ds(b0, BT)]

    def start_in(ci, b):
        pltpu.async_copy(in_src(ci), x_v.at[b], sem_in.at[b])

    def wait_in(b):
        pltpu.make_async_copy(in_src(0), x_v.at[b], sem_in.at[b]).wait()

    def start_out(ci, b):
        pltpu.async_copy(out_v.at[b], out_dst(ci), sem_out.at[b])

    def wait_out(b):
        pltpu.make_async_copy(out_v.at[b], out_dst(0), sem_out.at[b]).wait()

    def compute(xb, ob):
        @plsc.parallel_loop(0, NA * NGRP, 1, unroll=1)
        def group_body(i):
            a = i // NGRP
            g = i % NGRP
            bsl = pl.ds(g * L, L)
            items = [(x_v[xb, c, a, bsl], jnp.full((L,), c, jnp.int32))
                     for c in range(CN)]
            while len(items) > 1:
                nxt = []
                for j in range(0, len(items) - 1, 2):
                    lv, li = items[j]
                    rv, ri = items[j + 1]
                    gt = rv > lv
                    nxt.append((jnp.where(gt, rv, lv),
                                jnp.where(gt, ri, li)))
                if len(items) % 2:
                    nxt.append(items[-1])
                items = nxt
            best = items[0][1]
            base = best * OD + skew
            basr = [base + r for r in range(8)]
            vals = [
                plsc.load_gather(
                    ef_v.at[pl.ds(8 * (d // 8), L * EFS - OD + 8)],
                    [basr[d % 8]])
                for d in range(OD)]
            for d in range(OD):
                out_v[ob, a, d, bsl] = vals[d]

    def proc(ci, xb, ob, need_out_wait, may_prefetch):
        wait_in(xb)
        if need_out_wait:
            wait_out(ob)
        compute(xb, ob)
        start_out(ci, ob)
        if may_prefetch:
            start_in(ci + NXB, xb)

    for b in range(NXB):
        start_in(b, b)
    proc(0, 0, 0, False, True)
    proc(1, 1, 1, False, True)

    def loop_i(i, _):
        ci0 = 6 * i + 2
        for k in range(6):
            proc(ci0 + k, (2 + k) % NXB, k % NOB, True, True)
        return ()

    lax.fori_loop(0, 3, loop_i, ())
    for ci in range(20, NCHUNK):
        proc(ci, ci % NXB, ci % NOB, True, ci + NXB < NCHUNK)
    wait_out(0)
    wait_out(1)


@jax.jit
def kernel(x, emb):
    mesh = plsc.VectorSubcoreMesh(core_axis_name="c", subcore_axis_name="s")
    f = pl.kernel(
        _body,
        out_type=jax.ShapeDtypeStruct((AN, OD, BS), jnp.float32),
        mesh=mesh,
        scratch_types=[
            pltpu.VMEM((NXB, CN, NA, BT), jnp.float32),
            pltpu.VMEM((L * EFS,), jnp.float32),
            pltpu.VMEM((NOB, NA, OD, BT), jnp.float32),
            pltpu.SemaphoreType.DMA((NXB,)),
            pltpu.SemaphoreType.DMA((NOB,)),
        ],
        compiler_params=pltpu.CompilerParams(
            use_tc_tiling_on_sc=True, needs_layout_passes=False),
    )
    x_t = jnp.transpose(x, (2, 1, 0))
    out_t = f(x_t, emb)
    return jnp.transpose(out_t, (0, 2, 1))


# speedup vs baseline: 1.2088x; 1.2088x over previous
import jax
import jax.numpy as jnp
from jax import lax
from jax.experimental import pallas as pl
from jax.experimental.pallas import tpu as pltpu
from jax.experimental.pallas import tpu_sc as plsc

BS, AN, CN, OD = 4096, 200, 18, 32
NC, NS, L = 2, 16, 16
NW = NC * NS
BT = BS // NW
NA = 8
NCHUNK = AN // NA
NGRP = BT // L
EFS = CN * OD + 1


def _body(x_hbm, emb_hbm, out_hbm, x_v, emb_v, ef_v, out_v, sem_in, sem_out):
    wid = lax.axis_index("c") * NS + lax.axis_index("s")
    b0 = wid * BT

    pltpu.sync_copy(emb_hbm, emb_v)
    for l in range(L):
        for i in range(CN):
            for h in range(OD // L):
                ef_v[pl.ds(l * EFS + i * OD + h * L, L)] = (
                    emb_v[i, pl.ds(h * L, L)])
    skew = lax.iota(jnp.int32, L) * EFS

    def in_src(ci):
        return x_hbm.at[:, pl.ds(ci * NA, NA), pl.ds(b0, BT)]

    def out_dst(ci):
        return out_hbm.at[pl.ds(ci * NA, NA), :, pl.ds(b0, BT)]

    def start_in(ci, b):
        pltpu.async_copy(in_src(ci), x_v.at[b], sem_in.at[b])

    def wait_in(b):
        pltpu.make_async_copy(in_src(0), x_v.at[b], sem_in.at[b]).wait()

    def start_out(ci, b):
        pltpu.async_copy(out_v.at[b], out_dst(ci), sem_out.at[b])

    def wait_out(b):
        pltpu.make_async_copy(out_v.at[b], out_dst(0), sem_out.at[b]).wait()

    def compute(b):
        @plsc.parallel_loop(0, NA * NGRP, 1, unroll=1)
        def group_body(i):
            a = i // NGRP
            g = i % NGRP
            bsl = pl.ds(g * L, L)
            items = [(x_v[b, c, a, bsl], jnp.full((L,), c, jnp.int32))
                     for c in range(CN)]
            while len(items) > 1:
                nxt = []
                for j in range(0, len(items) - 1, 2):
                    lv, li = items[j]
                    rv, ri = items[j + 1]
                    gt = rv > lv
                    nxt.append((jnp.where(gt, rv, lv),
                                jnp.where(gt, ri, li)))
                if len(items) % 2:
                    nxt.append(items[-1])
                items = nxt
            best = items[0][1]
            base = best * OD + skew
            basr = [base + r for r in range(8)]
            vals = [
                plsc.load_gather(
                    ef_v.at[pl.ds(8 * (d // 8), L * EFS - OD + 8)],
                    [basr[d % 8]])
                for d in range(OD)]
            for d in range(OD):
                out_v[b, a, d, bsl] = vals[d]

    start_in(0, 0)
    start_in(1, 1)
    wait_in(0)
    compute(0)
    start_out(0, 0)
    start_in(2, 0)

    def loop_i(i, _):
        ci = 2 * i + 1
        wait_in(1)

        @pl.when(i > 0)
        def _():
            wait_out(1)

        compute(1)
        start_out(ci, 1)

        @pl.when(i < 11)
        def _():
            start_in(ci + 2, 1)

        wait_in(0)
        wait_out(0)
        compute(0)
        start_out(ci + 1, 0)

        @pl.when(i < 11)
        def _():
            start_in(ci + 3, 0)

        return ()

    lax.fori_loop(0, (NCHUNK - 1) // 2, loop_i, ())
    wait_out(1)
    wait_out(0)


@jax.jit
def kernel(x, emb):
    mesh = plsc.VectorSubcoreMesh(core_axis_name="c", subcore_axis_name="s")
    f = pl.kernel(
        _body,
        out_type=jax.ShapeDtypeStruct((AN, OD, BS), jnp.float32),
        mesh=mesh,
        scratch_types=[
            pltpu.VMEM((2, CN, NA, BT), jnp.float32),
            pltpu.VMEM((CN, OD), jnp.float32),
            pltpu.VMEM((L * EFS,), jnp.float32),
            pltpu.VMEM((2, NA, OD, BT), jnp.float32),
            pltpu.SemaphoreType.DMA((2,)),
            pltpu.SemaphoreType.DMA((2,)),
        ],
        compiler_params=pltpu.CompilerParams(
            use_tc_tiling_on_sc=True, needs_layout_passes=False),
    )
    x_t = jnp.transpose(x, (2, 1, 0))
    out_t = f(x_t, emb)
    return jnp.transpose(out_t, (0, 2, 1))
